# baseline (speedup 1.0000x reference)
import functools

import jax
import jax.numpy as jnp
from jax import lax
from jax.experimental import pallas as pl
from jax.experimental.pallas import tpu as pltpu
from jax.experimental.pallas import tpu_sc as plsc

CHUNK = 32
LANES = 16


@functools.lru_cache(maxsize=None)
def _make_lookup(B, V, D):
    info = plsc.get_sparse_core_info()
    NC, NS = info.num_cores, info.num_subcores
    NW = NC * NS
    assert B % (NW * CHUNK) == 0
    b_per_w = B // NW
    n = b_per_w // CHUNK
    assert n % 2 == 0 and n >= 4

    mesh = plsc.VectorSubcoreMesh(core_axis_name="c", subcore_axis_name="s")

    @functools.partial(
        pl.kernel,
        mesh=mesh,
        out_type=jax.ShapeDtypeStruct((B, D), jnp.float32),
        scratch_types=[
            pltpu.VMEM((V, D), jnp.float32),
            pltpu.VMEM((n, CHUNK), jnp.int32),
            pltpu.VMEM_SHARED((NS, 2, CHUNK, D), jnp.float32),
            pltpu.VMEM((CHUNK, D), jnp.float32),
            pltpu.VMEM((CHUNK, D), jnp.float32),
            pltpu.SemaphoreType.DMA,
            pltpu.SemaphoreType.DMA,
            pltpu.SemaphoreType.DMA,
            pltpu.SemaphoreType.DMA,
        ],
    )
    def lookup(idx_hbm, table_hbm, out_hbm, table_v, idx_v, spm, buf0, buf1,
               ts0, ts1, sh0, sh1):
        bufs = (buf0, buf1)
        tsems = (ts0, ts1)
        hsems = (sh0, sh1)

        sid = lax.axis_index("s")
        wid = sid * NC + lax.axis_index("c")
        base = wid * b_per_w
        pltpu.sync_copy(table_hbm, table_v)
        pltpu.sync_copy(idx_hbm.at[wid], idx_v)

        def compute(g, b):
            def quarter(k, carry):
                iv = idx_v[g, pl.ds(k * LANES, LANES)]
                for l in range(LANES):
                    i = iv[l]
                    r = k * LANES + l
                    for c in range(D // LANES):
                        sl = pl.ds(c * LANES, LANES)
                        bufs[b][r, sl] = table_v[i, sl]
                return carry

            lax.fori_loop(0, CHUNK // LANES, quarter, 0)

        def push_to_spmem(p):
            pltpu.async_copy(bufs[p], spm.at[sid, p], tsems[p])

        def wait_to_spmem(p):
            pltpu.make_async_copy(bufs[p], spm.at[sid, p], tsems[p]).wait()

        def push_to_hbm(g, p):
            pltpu.async_copy(
                spm.at[sid, p], out_hbm.at[pl.ds(base + g * CHUNK, CHUNK)],
                hsems[p],
            )

        def wait_to_hbm(g, p):
            pltpu.make_async_copy(
                spm.at[sid, p], out_hbm.at[pl.ds(base + g * CHUNK, CHUNK)],
                hsems[p],
            ).wait()

        def step(g, p, first):
            if not first:
                wait_to_hbm(g - 2, p)
            compute(g, p)
            push_to_spmem(p)
            wait_to_spmem(p)
            push_to_hbm(g, p)

        for g in (0, 1):
            step(g, g, True)

        def pair(q, carry):
            for j in range(2):
                step(2 * q + j, j, False)
            return carry

        lax.fori_loop(1, n // 2, pair, 0)

        wait_to_hbm(n - 2, 0)
        wait_to_hbm(n - 1, 1)

    return lookup


def kernel(input_ids, embed_weight):
    B = input_ids.shape[0] * input_ids.shape[1]
    V, D = embed_weight.shape
    info = plsc.get_sparse_core_info()
    NW = info.num_cores * info.num_subcores
    idx = input_ids.reshape(NW, (B // NW) // CHUNK, CHUNK).astype(jnp.int32)
    out = _make_lookup(B, V, D)(idx, embed_weight)
    return out.reshape(input_ids.shape[0], input_ids.shape[1], D)


# speedup vs baseline: 1.0361x; 1.0361x over previous
import functools

import jax
import jax.numpy as jnp
from jax import lax
from jax.experimental import pallas as pl
from jax.experimental.pallas import tpu as pltpu
from jax.experimental.pallas import tpu_sc as plsc

CHUNK = 32
LANES = 16


@functools.lru_cache(maxsize=None)
def _make_lookup(B, V, D):
    info = plsc.get_sparse_core_info()
    NC, NS = info.num_cores, info.num_subcores
    NW = NC * NS
    assert B % (NW * CHUNK) == 0
    b_per_w = B // NW
    n = b_per_w // CHUNK
    assert n % 4 == 0 and n >= 8

    mesh = plsc.VectorSubcoreMesh(core_axis_name="c", subcore_axis_name="s")

    @functools.partial(
        pl.kernel,
        mesh=mesh,
        out_type=jax.ShapeDtypeStruct((B, D), jnp.float32),
        scratch_types=[
            pltpu.VMEM((V, D), jnp.float32),
            pltpu.VMEM((n, CHUNK), jnp.int32),
            pltpu.VMEM_SHARED((NS, 1, CHUNK, D), jnp.float32),
        ]
        + [pltpu.VMEM((CHUNK, D), jnp.float32) for _ in range(3)]
        + [pltpu.SemaphoreType.DMA for _ in range(6)],
    )
    def lookup(idx_hbm, table_hbm, out_hbm, table_v, idx_v, spm,
               db0, db1, sb, dsem0, dsem1, tsem, hsem, xsem0, xsem1):
        dbufs = (db0, db1)
        dsems = (dsem0, dsem1)

        sid = lax.axis_index("s")
        wid = sid * NC + lax.axis_index("c")
        base = wid * b_per_w
        pltpu.sync_copy(table_hbm, table_v)
        pltpu.sync_copy(idx_hbm.at[wid], idx_v)

        def compute(g, buf):
            def block(k, carry):
                iv = idx_v[g, pl.ds(k * LANES, LANES)]
                for l in range(LANES):
                    i = iv[l]
                    r = k * LANES + l
                    for c in range(D // LANES):
                        sl = pl.ds(c * LANES, LANES)
                        buf[r, sl] = table_v[i, sl]
                return carry

            lax.fori_loop(0, CHUNK // LANES, block, 0)

        def out_slice(g):
            return out_hbm.at[pl.ds(base + g * CHUNK, CHUNK)]

        def step_direct(g, b, first):
            if not first:
                pltpu.make_async_copy(dbufs[b], out_slice(g - 4), dsems[b]).wait()
            compute(g, dbufs[b])
            pltpu.async_copy(dbufs[b], out_slice(g), dsems[b])

        def step_spmem(g, first):
            if not first:
                pltpu.make_async_copy(
                    spm.at[sid, 0], out_slice(g - 2), hsem
                ).wait()
            compute(g, sb)
            pltpu.async_copy(sb, spm.at[sid, 0], tsem)
            pltpu.make_async_copy(sb, spm.at[sid, 0], tsem).wait()
            pltpu.async_copy(spm.at[sid, 0], out_slice(g), hsem)

        def quad(q, first):
            g = 4 * q
            step_direct(g, 0, first)
            step_spmem(g + 1, first)
            step_direct(g + 2, 1, first)
            step_spmem(g + 3, False)

        quad(0, True)

        def body(q, carry):
            quad(q, False)
            return carry

        lax.fori_loop(1, n // 4, body, 0)

        pltpu.make_async_copy(dbufs[0], out_slice(n - 4), dsems[0]).wait()
        pltpu.make_async_copy(dbufs[1], out_slice(n - 2), dsems[1]).wait()
        pltpu.make_async_copy(spm.at[sid, 0], out_slice(n - 1), hsem).wait()

    return lookup


def kernel(input_ids, embed_weight):
    B = input_ids.shape[0] * input_ids.shape[1]
    V, D = embed_weight.shape
    info = plsc.get_sparse_core_info()
    NW = info.num_cores * info.num_subcores
    idx = input_ids.reshape(NW, (B // NW) // CHUNK, CHUNK).astype(jnp.int32)
    out = _make_lookup(B, V, D)(idx, embed_weight)
    return out.reshape(input_ids.shape[0], input_ids.shape[1], D)


# speedup vs baseline: 1.5075x; 1.4550x over previous
import functools

import jax
import jax.numpy as jnp
from jax import lax
from jax.experimental import pallas as pl
from jax.experimental.pallas import tpu as pltpu
from jax.experimental.pallas import tpu_sc as plsc

LANES = 16


@functools.lru_cache(maxsize=None)
def _make_lookup(S, T, V, D):
    info = plsc.get_sparse_core_info()
    NC, NS = info.num_cores, info.num_subcores
    NW = NC * NS
    assert S % NW == 0
    s_per_w = S // NW
    assert s_per_w % 2 == 0
    TP = (T + LANES - 1) // LANES * LANES

    mesh = plsc.VectorSubcoreMesh(core_axis_name="c", subcore_axis_name="s")

    @functools.partial(
        pl.kernel,
        mesh=mesh,
        out_type=jax.ShapeDtypeStruct((S, T, D), jnp.float32),
        scratch_types=[
            pltpu.VMEM((V, D), jnp.float32),
            pltpu.VMEM((s_per_w, TP), jnp.int32),
            pltpu.VMEM((T, D), jnp.float32),
            pltpu.VMEM((T, D), jnp.float32),
            pltpu.SemaphoreType.DMA,
            pltpu.SemaphoreType.DMA,
        ],
    )
    def lookup(idx_hbm, table_hbm, out_hbm, table_v, idx_v, buf0, buf1,
               sem0, sem1):
        bufs = (buf0, buf1)
        sems = (sem0, sem1)

        wid = lax.axis_index("s") * NC + lax.axis_index("c")
        base = wid * s_per_w
        pltpu.sync_copy(table_hbm, table_v)
        pltpu.sync_copy(idx_hbm.at[wid], idx_v)

        def expand_rows(g, buf, k, nrows):
            iv = idx_v[g, pl.ds(k * LANES, LANES)]
            for l in range(nrows):
                i = iv[l]
                r = k * LANES + l
                for c in range(D // LANES):
                    sl = pl.ds(c * LANES, LANES)
                    buf[r, sl] = table_v[i, sl]

        def compute(g, buf):
            def block(k, carry):
                expand_rows(g, buf, k, LANES)
                return carry

            lax.fori_loop(0, T // LANES, block, 0)
            if T % LANES:
                expand_rows(g, buf, T // LANES, T % LANES)

        def start_write(g, b):
            pltpu.async_copy(bufs[b], out_hbm.at[base + g], sems[b])

        def wait_write(g, b):
            pltpu.make_async_copy(bufs[b], out_hbm.at[base + g], sems[b]).wait()

        for g in (0, 1):
            compute(g, bufs[g])
            start_write(g, g)

        def pair(q, carry):
            for j in range(2):
                g = 2 * q + j
                wait_write(g - 2, j)
                compute(g, bufs[j])
                start_write(g, j)
            return carry

        lax.fori_loop(1, s_per_w // 2, pair, 0)

        wait_write(s_per_w - 2, 0)
        wait_write(s_per_w - 1, 1)

    return lookup


def kernel(input_ids, embed_weight):
    S, T = input_ids.shape
    V, D = embed_weight.shape
    info = plsc.get_sparse_core_info()
    NW = info.num_cores * info.num_subcores
    idx = input_ids.reshape(NW, S // NW, T).astype(jnp.int32)
    TP = (T + LANES - 1) // LANES * LANES
    if TP != T:
        idx = jnp.pad(idx, ((0, 0), (0, 0), (0, TP - T)))
    return _make_lookup(S, T, V, D)(idx, embed_weight)
